# Initial kernel scaffold; baseline (speedup 1.0000x reference)
import jax
import jax.numpy as jnp
from jax.experimental import pallas as pl


def kernel(points, spoints):
    raise NotImplementedError("write your pallas kernel here")


# speedup vs baseline: 18.7352x; 18.7352x over previous
import dataclasses
import functools

import jax
import jax.numpy as jnp
from jax import lax
from jax.experimental import pallas as pl
from jax.experimental.pallas import tpu as pltpu
from jax.experimental.pallas import tpu_sc as plsc

_K = 11
_BN = 512
_NW = 32


def _topk_body(p_ref, st_ref, idx_ref):
    p = p_ref[0]
    st = st_ref[0]
    m = st.shape[1]
    bn = p.shape[0]
    dist = None
    for d in range(3):
        diff = p[:, d:d + 1] - st[d:d + 1, :]
        dist = diff * diff if dist is None else dist + diff * diff
    iota = lax.broadcasted_iota(jnp.int32, (bn, m), 1)
    inf = jnp.float32(jnp.inf)
    work = dist
    for k in range(_K):
        mn = jnp.min(work, axis=1, keepdims=True)
        eq = work == mn
        idxk = jnp.min(jnp.where(eq, iota, m), axis=1, keepdims=True)
        idx_ref[0, :, k:k + 1] = idxk
        if k < _K - 1:
            work = jnp.where(eq, inf, work)


def _topk_indices(points, spoints_t, interpret=False):
    b, n, _ = points.shape
    m = spoints_t.shape[2]
    return pl.pallas_call(
        _topk_body,
        grid=(b, n // _BN),
        in_specs=[
            pl.BlockSpec((1, _BN, 3), lambda bi, i: (bi, i, 0)),
            pl.BlockSpec((1, 3, m), lambda bi, i: (bi, 0, 0)),
        ],
        out_specs=pl.BlockSpec((1, _BN, _K), lambda bi, i: (bi, i, 0)),
        out_shape=jax.ShapeDtypeStruct((b, n, _K), jnp.int32),
        interpret=interpret,
    )(points, spoints_t)


def _edge_min(idx_w, points_t, spoints_t):
    b, _, n = points_t.shape
    m = spoints_t.shape[2]
    total = b * n
    ppw = total // _NW
    wpb = _NW // b
    mesh = plsc.VectorSubcoreMesh(core_axis_name="c", subcore_axis_name="s")
    cp = pltpu.CompilerParams()
    if "needs_layout_passes" in pltpu.CompilerParams.__dataclass_fields__:
        cp = dataclasses.replace(cp, needs_layout_passes=False)

    @functools.partial(
        pl.kernel,
        out_type=jax.ShapeDtypeStruct((total,), jnp.float32),
        mesh=mesh,
        compiler_params=cp,
        scratch_types=[
            pltpu.VMEM((3, m), jnp.float32),
            pltpu.VMEM((3, ppw), jnp.float32),
            pltpu.VMEM((_K, ppw), jnp.int32),
            pltpu.VMEM((ppw,), jnp.float32),
        ],
    )
    def body(idx_hbm, p_hbm, s_hbm, out_hbm, sv, pv, iv, ov):
        wid = lax.axis_index("s") * 2 + lax.axis_index("c")
        bi = wid // wpb
        nbase = (wid % wpb) * ppw
        base = wid * ppw
        pltpu.sync_copy(s_hbm.at[bi], sv)
        pltpu.sync_copy(p_hbm.at[bi, :, pl.ds(nbase, ppw)], pv)
        pltpu.sync_copy(idx_hbm.at[wid], iv)

        row0 = jnp.zeros((16,), jnp.int32)
        row1 = jnp.full((16,), 1, jnp.int32)
        row2 = jnp.full((16,), 2, jnp.int32)

        @pl.loop(0, ppw, step=16)
        def _chunk(c):
            sl = pl.ds(c, 16)
            pxv, pyv, pzv = pv[0, sl], pv[1, sl], pv[2, sl]
            i0 = iv[0, sl]
            c0x = plsc.load_gather(sv, [row0, i0])
            c0y = plsc.load_gather(sv, [row1, i0])
            c0z = plsc.load_gather(sv, [row2, i0])
            d0x, d0y, d0z = pxv - c0x, pyv - c0y, pzv - c0z
            d0 = d0x * d0x + d0y * d0y + d0z * d0z
            acc = jnp.full((16,), jnp.inf, jnp.float32)
            for j in range(1, _K):
                ij = iv[j, sl]
                cjx = plsc.load_gather(sv, [row0, ij])
                cjy = plsc.load_gather(sv, [row1, ij])
                cjz = plsc.load_gather(sv, [row2, ij])
                ex, ey, ez = cjx - c0x, cjy - c0y, cjz - c0z
                lsq = ex * ex + ey * ey + ez * ez
                qx, qy, qz = pxv - cjx, pyv - cjy, pzv - cjz
                dj = qx * qx + qy * qy + qz * qz
                diff = dj - d0
                val = (diff * diff) / (4.0 * lsq)
                acc = jnp.minimum(acc, val)
            ov[sl] = acc

        pltpu.sync_copy(ov, out_hbm.at[pl.ds(base, ppw)])

    return body(idx_w, points_t, spoints_t)


def kernel(points, spoints):
    b, n, _ = points.shape
    total = b * n
    ppw = total // _NW
    spoints_t = jnp.transpose(spoints, (0, 2, 1))
    idx = _topk_indices(points, spoints_t)
    idx_w = idx.reshape(_NW, ppw, _K).transpose(0, 2, 1)
    points_t = jnp.transpose(points, (0, 2, 1))
    out = _edge_min(idx_w, points_t, spoints_t)
    return out.reshape(b, n)


# speedup vs baseline: 22.0350x; 1.1761x over previous
import dataclasses
import functools

import jax
import jax.numpy as jnp
from jax import lax
from jax.experimental import pallas as pl
from jax.experimental.pallas import tpu as pltpu
from jax.experimental.pallas import tpu_sc as plsc

_K = 11
_BN = 512
_NW = 32


def _topk_body(p_ref, st_ref, idx_ref):
    p = p_ref[0]
    st = st_ref[0]
    m = st.shape[1]
    bn = p.shape[0]
    nc = m // 128
    dist = None
    for d in range(3):
        diff = p[:, d:d + 1] - st[d:d + 1, :]
        dist = diff * diff if dist is None else dist + diff * diff
    bits = lax.bitcast_convert_type(dist, jnp.int32)
    chunks = [
        (bits[:, c * 128:(c + 1) * 128] & jnp.int32(~15)) | jnp.int32(c)
        for c in range(nc)
    ]
    lane_iota = lax.broadcasted_iota(jnp.int32, (bn, 128), 1)
    imax = jnp.int32(2147483647)
    prev = jnp.full((bn, 1), -1, jnp.int32)
    for k in range(_K):
        acc = None
        for c in range(nc):
            cand = jnp.where(chunks[c] > prev, chunks[c], imax)
            acc = cand if acc is None else jnp.minimum(acc, cand)
        mn = jnp.min(acc, axis=1, keepdims=True)
        lane = jnp.min(jnp.where(acc == mn, lane_iota, 128),
                       axis=1, keepdims=True)
        idx_ref[0, :, k:k + 1] = (mn & 15) * 128 + lane
        prev = mn


def _topk_indices(points, spoints_t, interpret=False):
    b, n, _ = points.shape
    m = spoints_t.shape[2]
    return pl.pallas_call(
        _topk_body,
        grid=(b, n // _BN),
        in_specs=[
            pl.BlockSpec((1, _BN, 3), lambda bi, i: (bi, i, 0)),
            pl.BlockSpec((1, 3, m), lambda bi, i: (bi, 0, 0)),
        ],
        out_specs=pl.BlockSpec((1, _BN, _K), lambda bi, i: (bi, i, 0)),
        out_shape=jax.ShapeDtypeStruct((b, n, _K), jnp.int32),
        interpret=interpret,
    )(points, spoints_t)


def _edge_min(idx_w, points_t, spoints_t):
    b, _, n = points_t.shape
    m = spoints_t.shape[2]
    total = b * n
    ppw = total // _NW
    wpb = _NW // b
    mesh = plsc.VectorSubcoreMesh(core_axis_name="c", subcore_axis_name="s")
    cp = pltpu.CompilerParams()
    if "needs_layout_passes" in pltpu.CompilerParams.__dataclass_fields__:
        cp = dataclasses.replace(cp, needs_layout_passes=False)

    @functools.partial(
        pl.kernel,
        out_type=jax.ShapeDtypeStruct((total,), jnp.float32),
        mesh=mesh,
        compiler_params=cp,
        scratch_types=[
            pltpu.VMEM((3, m), jnp.float32),
            pltpu.VMEM((3, ppw), jnp.float32),
            pltpu.VMEM((_K, ppw), jnp.int32),
            pltpu.VMEM((ppw,), jnp.float32),
        ],
    )
    def body(idx_hbm, p_hbm, s_hbm, out_hbm, sv, pv, iv, ov):
        wid = lax.axis_index("s") * 2 + lax.axis_index("c")
        bi = wid // wpb
        nbase = (wid % wpb) * ppw
        base = wid * ppw
        pltpu.sync_copy(s_hbm.at[bi], sv)
        pltpu.sync_copy(p_hbm.at[bi, :, pl.ds(nbase, ppw)], pv)
        pltpu.sync_copy(idx_hbm.at[wid], iv)

        row0 = jnp.zeros((16,), jnp.int32)
        row1 = jnp.full((16,), 1, jnp.int32)
        row2 = jnp.full((16,), 2, jnp.int32)

        @pl.loop(0, ppw, step=16)
        def _chunk(c):
            sl = pl.ds(c, 16)
            pxv, pyv, pzv = pv[0, sl], pv[1, sl], pv[2, sl]
            i0 = iv[0, sl]
            c0x = plsc.load_gather(sv, [row0, i0])
            c0y = plsc.load_gather(sv, [row1, i0])
            c0z = plsc.load_gather(sv, [row2, i0])
            d0x, d0y, d0z = pxv - c0x, pyv - c0y, pzv - c0z
            d0 = d0x * d0x + d0y * d0y + d0z * d0z
            acc = jnp.full((16,), jnp.inf, jnp.float32)
            for j in range(1, _K):
                ij = iv[j, sl]
                cjx = plsc.load_gather(sv, [row0, ij])
                cjy = plsc.load_gather(sv, [row1, ij])
                cjz = plsc.load_gather(sv, [row2, ij])
                ex, ey, ez = cjx - c0x, cjy - c0y, cjz - c0z
                lsq = ex * ex + ey * ey + ez * ez
                qx, qy, qz = pxv - cjx, pyv - cjy, pzv - cjz
                dj = qx * qx + qy * qy + qz * qz
                diff = dj - d0
                val = (diff * diff) / (4.0 * lsq)
                acc = jnp.minimum(acc, val)
            ov[sl] = acc

        pltpu.sync_copy(ov, out_hbm.at[pl.ds(base, ppw)])

    return body(idx_w, points_t, spoints_t)


def kernel(points, spoints):
    b, n, _ = points.shape
    total = b * n
    ppw = total // _NW
    spoints_t = jnp.transpose(spoints, (0, 2, 1))
    idx = _topk_indices(points, spoints_t)
    idx_w = idx.reshape(_NW, ppw, _K).transpose(0, 2, 1)
    points_t = jnp.transpose(points, (0, 2, 1))
    out = _edge_min(idx_w, points_t, spoints_t)
    return out.reshape(b, n)


# speedup vs baseline: 22.5196x; 1.0220x over previous
import dataclasses
import functools

import jax
import jax.numpy as jnp
from jax import lax
from jax.experimental import pallas as pl
from jax.experimental.pallas import tpu as pltpu
from jax.experimental.pallas import tpu_sc as plsc

_K = 11
_BN = 512
_NW = 32


def _topk_body(p_ref, st_ref, idx_ref):
    p = p_ref[0]
    st = st_ref[0]
    m = st.shape[1]
    bn = p.shape[0]
    nc = m // 128
    dist = None
    for d in range(3):
        diff = p[:, d:d + 1] - st[d:d + 1, :]
        dist = diff * diff if dist is None else dist + diff * diff
    bits = lax.bitcast_convert_type(dist, jnp.int32)
    chunks = [
        (bits[:, c * 128:(c + 1) * 128] & jnp.int32(~15)) | jnp.int32(c)
        for c in range(nc)
    ]
    lane_iota = lax.broadcasted_iota(jnp.int32, (bn, 128), 1)
    imax = jnp.int32(2147483647)
    prev = None
    for k in range(_K):
        acc = None
        for c in range(nc):
            cand = (chunks[c] if prev is None
                    else jnp.where(chunks[c] > prev, chunks[c], imax))
            acc = cand if acc is None else jnp.minimum(acc, cand)
        mn = jnp.min(acc, axis=1, keepdims=True)
        lane = jnp.min(jnp.where(acc == mn, lane_iota, 128),
                       axis=1, keepdims=True)
        idx_ref[0, :, k:k + 1] = (mn & 15) * 128 + lane
        prev = mn


def _topk_indices(points, spoints_t, interpret=False):
    b, n, _ = points.shape
    m = spoints_t.shape[2]
    return pl.pallas_call(
        _topk_body,
        grid=(b, n // _BN),
        in_specs=[
            pl.BlockSpec((1, _BN, 3), lambda bi, i: (bi, i, 0)),
            pl.BlockSpec((1, 3, m), lambda bi, i: (bi, 0, 0)),
        ],
        out_specs=pl.BlockSpec((1, _BN, _K), lambda bi, i: (bi, i, 0)),
        out_shape=jax.ShapeDtypeStruct((b, n, _K), jnp.int32),
        interpret=interpret,
    )(points, spoints_t)


def _edge_min(idx_w, points_t, spoints_t):
    b, _, n = points_t.shape
    m = spoints_t.shape[2]
    total = b * n
    ppw = total // _NW
    wpb = _NW // b
    mesh = plsc.VectorSubcoreMesh(core_axis_name="c", subcore_axis_name="s")
    cp = pltpu.CompilerParams()
    if "needs_layout_passes" in pltpu.CompilerParams.__dataclass_fields__:
        cp = dataclasses.replace(cp, needs_layout_passes=False)

    @functools.partial(
        pl.kernel,
        out_type=jax.ShapeDtypeStruct((total,), jnp.float32),
        mesh=mesh,
        compiler_params=cp,
        scratch_types=[
            pltpu.VMEM((3, m), jnp.float32),
            pltpu.VMEM((3, ppw), jnp.float32),
            pltpu.VMEM((_K, ppw), jnp.int32),
            pltpu.VMEM((ppw,), jnp.float32),
        ],
    )
    def body(idx_hbm, p_hbm, s_hbm, out_hbm, sv, pv, iv, ov):
        wid = lax.axis_index("s") * 2 + lax.axis_index("c")
        bi = wid // wpb
        nbase = (wid % wpb) * ppw
        base = wid * ppw
        pltpu.sync_copy(s_hbm.at[bi], sv)
        pltpu.sync_copy(p_hbm.at[bi, :, pl.ds(nbase, ppw)], pv)
        pltpu.sync_copy(idx_hbm.at[wid], iv)

        row0 = jnp.zeros((16,), jnp.int32)
        row1 = jnp.full((16,), 1, jnp.int32)
        row2 = jnp.full((16,), 2, jnp.int32)

        @pl.loop(0, ppw, step=16)
        def _chunk(c):
            sl = pl.ds(c, 16)
            pxv, pyv, pzv = pv[0, sl], pv[1, sl], pv[2, sl]
            i0 = iv[0, sl]
            c0x = plsc.load_gather(sv, [row0, i0])
            c0y = plsc.load_gather(sv, [row1, i0])
            c0z = plsc.load_gather(sv, [row2, i0])
            d0x, d0y, d0z = pxv - c0x, pyv - c0y, pzv - c0z
            d0 = d0x * d0x + d0y * d0y + d0z * d0z
            acc = jnp.full((16,), jnp.inf, jnp.float32)
            for j in range(1, _K):
                ij = iv[j, sl]
                cjx = plsc.load_gather(sv, [row0, ij])
                cjy = plsc.load_gather(sv, [row1, ij])
                cjz = plsc.load_gather(sv, [row2, ij])
                ex, ey, ez = cjx - c0x, cjy - c0y, cjz - c0z
                lsq = ex * ex + ey * ey + ez * ez
                qx, qy, qz = pxv - cjx, pyv - cjy, pzv - cjz
                dj = qx * qx + qy * qy + qz * qz
                diff = dj - d0
                val = (diff * diff) / (4.0 * lsq)
                acc = jnp.minimum(acc, val)
            ov[sl] = acc

        pltpu.sync_copy(ov, out_hbm.at[pl.ds(base, ppw)])

    return body(idx_w, points_t, spoints_t)


def kernel(points, spoints):
    b, n, _ = points.shape
    total = b * n
    ppw = total // _NW
    spoints_t = jnp.transpose(spoints, (0, 2, 1))
    idx = _topk_indices(points, spoints_t)
    idx_w = idx.reshape(_NW, ppw, _K).transpose(0, 2, 1)
    points_t = jnp.transpose(points, (0, 2, 1))
    out = _edge_min(idx_w, points_t, spoints_t)
    return out.reshape(b, n)


# speedup vs baseline: 23.3325x; 1.0361x over previous
import dataclasses
import functools

import jax
import jax.numpy as jnp
from jax import lax
from jax.experimental import pallas as pl
from jax.experimental.pallas import tpu as pltpu
from jax.experimental.pallas import tpu_sc as plsc

_K = 11
_BN = 1024
_NW = 32


def _topk_body(p_ref, st_ref, idx_ref):
    p = p_ref[0]
    st = st_ref[0]
    m = st.shape[1]
    bn = p.shape[0]
    nc = m // 128
    dist = None
    for d in range(3):
        diff = p[:, d:d + 1] - st[d:d + 1, :]
        dist = diff * diff if dist is None else dist + diff * diff
    bits = lax.bitcast_convert_type(dist, jnp.int32)
    chunks = [
        (bits[:, c * 128:(c + 1) * 128] & jnp.int32(~15)) | jnp.int32(c)
        for c in range(nc)
    ]
    lane_iota = lax.broadcasted_iota(jnp.int32, (bn, 128), 1)
    imax = jnp.int32(2147483647)
    prev = None
    for k in range(_K):
        acc = None
        for c in range(nc):
            cand = (chunks[c] if prev is None
                    else jnp.where(chunks[c] > prev, chunks[c], imax))
            acc = cand if acc is None else jnp.minimum(acc, cand)
        mn = jnp.min(acc, axis=1, keepdims=True)
        lane = jnp.min(jnp.where(acc == mn, lane_iota, 128),
                       axis=1, keepdims=True)
        idx_ref[0, :, k:k + 1] = (mn & 15) * 128 + lane
        prev = mn


def _topk_indices(points, spoints_t, interpret=False):
    b, n, _ = points.shape
    m = spoints_t.shape[2]
    return pl.pallas_call(
        _topk_body,
        grid=(b, n // _BN),
        in_specs=[
            pl.BlockSpec((1, _BN, 3), lambda bi, i: (bi, i, 0)),
            pl.BlockSpec((1, 3, m), lambda bi, i: (bi, 0, 0)),
        ],
        out_specs=pl.BlockSpec((1, _BN, _K), lambda bi, i: (bi, i, 0)),
        out_shape=jax.ShapeDtypeStruct((b, n, _K), jnp.int32),
        interpret=interpret,
    )(points, spoints_t)


def _edge_min(idx_w, points_t, spoints_t):
    b, _, n = points_t.shape
    m = spoints_t.shape[2]
    total = b * n
    ppw = total // _NW
    wpb = _NW // b
    mesh = plsc.VectorSubcoreMesh(core_axis_name="c", subcore_axis_name="s")
    cp = pltpu.CompilerParams()
    if "needs_layout_passes" in pltpu.CompilerParams.__dataclass_fields__:
        cp = dataclasses.replace(cp, needs_layout_passes=False)

    @functools.partial(
        pl.kernel,
        out_type=jax.ShapeDtypeStruct((total,), jnp.float32),
        mesh=mesh,
        compiler_params=cp,
        scratch_types=[
            pltpu.VMEM((3, m), jnp.float32),
            pltpu.VMEM((3, ppw), jnp.float32),
            pltpu.VMEM((_K, ppw), jnp.int32),
            pltpu.VMEM((ppw,), jnp.float32),
        ],
    )
    def body(idx_hbm, p_hbm, s_hbm, out_hbm, sv, pv, iv, ov):
        wid = lax.axis_index("s") * 2 + lax.axis_index("c")
        bi = wid // wpb
        nbase = (wid % wpb) * ppw
        base = wid * ppw
        pltpu.sync_copy(s_hbm.at[bi], sv)
        pltpu.sync_copy(p_hbm.at[bi, :, pl.ds(nbase, ppw)], pv)
        pltpu.sync_copy(idx_hbm.at[wid], iv)

        row0 = jnp.zeros((16,), jnp.int32)
        row1 = jnp.full((16,), 1, jnp.int32)
        row2 = jnp.full((16,), 2, jnp.int32)

        @pl.loop(0, ppw, step=16)
        def _chunk(c):
            sl = pl.ds(c, 16)
            pxv, pyv, pzv = pv[0, sl], pv[1, sl], pv[2, sl]
            i0 = iv[0, sl]
            c0x = plsc.load_gather(sv, [row0, i0])
            c0y = plsc.load_gather(sv, [row1, i0])
            c0z = plsc.load_gather(sv, [row2, i0])
            d0x, d0y, d0z = pxv - c0x, pyv - c0y, pzv - c0z
            d0 = d0x * d0x + d0y * d0y + d0z * d0z
            acc = jnp.full((16,), jnp.inf, jnp.float32)
            for j in range(1, _K):
                ij = iv[j, sl]
                cjx = plsc.load_gather(sv, [row0, ij])
                cjy = plsc.load_gather(sv, [row1, ij])
                cjz = plsc.load_gather(sv, [row2, ij])
                ex, ey, ez = cjx - c0x, cjy - c0y, cjz - c0z
                lsq = ex * ex + ey * ey + ez * ez
                qx, qy, qz = pxv - cjx, pyv - cjy, pzv - cjz
                dj = qx * qx + qy * qy + qz * qz
                diff = dj - d0
                val = (diff * diff) / (4.0 * lsq)
                acc = jnp.minimum(acc, val)
            ov[sl] = acc

        pltpu.sync_copy(ov, out_hbm.at[pl.ds(base, ppw)])

    return body(idx_w, points_t, spoints_t)


def kernel(points, spoints):
    b, n, _ = points.shape
    total = b * n
    ppw = total // _NW
    spoints_t = jnp.transpose(spoints, (0, 2, 1))
    idx = _topk_indices(points, spoints_t)
    idx_w = idx.reshape(_NW, ppw, _K).transpose(0, 2, 1)
    points_t = jnp.transpose(points, (0, 2, 1))
    out = _edge_min(idx_w, points_t, spoints_t)
    return out.reshape(b, n)
